# baseline (speedup 1.0000x reference)
import functools

import jax
import jax.numpy as jnp
from jax import lax
from jax.experimental import pallas as pl
from jax.experimental.pallas import tpu as pltpu
from jax.experimental.pallas import tpu_sc as plsc

EMBED = 64
HIST = 50
HPAD = 64
HGAT = 56
NLANE = 16
LPAD = 16

NW = 32
NBUF = 8

NCOLB = 8
VCHUNK = 131072
MBLK = 8192


def _pmat_body(*refs):
    lhs = refs[:NCOLB]
    wb_ref, o_ref = refs[NCOLB], refs[NCOLB + 1]
    cat = jnp.concatenate([lhs[j][...] for j in range(NCOLB)], axis=0)
    o_ref[...] = lax.dot_general(
        cat, wb_ref[...], (((0,), (0,)), ((), ())),
        preferred_element_type=jnp.float32,
    )


def _pmat(table_t, w_big):
    grid = VCHUNK // MBLK
    nblk = table_t.shape[1] // MBLK
    in_specs = [
        pl.BlockSpec((EMBED, MBLK),
                     functools.partial(
                         lambda i, j: (0, jnp.minimum((VCHUNK // MBLK) * j + i,
                                                      nblk)),
                         j=j))
        for j in range(NCOLB)
    ]
    in_specs.append(
        pl.BlockSpec((NCOLB * EMBED, NCOLB * LPAD), lambda i: (0, 0)))
    return pl.pallas_call(
        _pmat_body,
        grid=(grid,),
        in_specs=in_specs,
        out_specs=pl.BlockSpec((MBLK, NCOLB * LPAD), lambda i: (i, 0)),
        out_shape=jax.ShapeDtypeStruct((VCHUNK, NCOLB * LPAD), jnp.float32),
    )(*([table_t] * NCOLB), w_big)


def _pool_body(p_hbm, batch_hbm, bias_hbm, out_hbm, idx_v, bufs, out_v,
               bias_v, sems, *, bpw):
    qpw = bpw // 2
    wid = lax.axis_index("s") * 2 + lax.axis_index("c")
    base = wid * bpw

    pltpu.sync_copy(batch_hbm.at[pl.ds(wid * qpw, qpw)], idx_v)
    pltpu.sync_copy(bias_hbm, bias_v)

    lanes = lax.iota(jnp.int32, NLANE)
    nj = HPAD // NLANE
    real = HIST - (HPAD - NLANE)

    def remap(q, carry):
        r0 = None
        for j in range(2 * nj):
            v = idx_v[q, pl.ds(j * NLANE, NLANE)]
            r = ((v & (VCHUNK - 1)) << 3) | (v >> 17)
            if j % nj == 0:
                r0 = r
            if j % nj == nj - 1:
                r = jnp.where(lanes < real, r, r0)
            idx_v[q, pl.ds(j * NLANE, NLANE)] = r
        return carry

    lax.fori_loop(0, qpw, remap, 0)

    def _gather(q, k):
        pltpu.make_async_copy(
            p_hbm.at[idx_v.at[q]], bufs.at[k], sems.at[k]).start()

    def _wait(q, k):
        pltpu.make_async_copy(
            p_hbm.at[idx_v.at[q]], bufs.at[k], sems.at[k]).wait()

    for k in range(NBUF):
        _gather(k, k)

    bias = bias_v[...]

    def g_body(g, carry):
        for k in range(NBUF):
            q = g * NBUF + k
            _wait(q, k)
            for t in range(2):
                acc = bufs[k, t * HPAD, :]
                for h in range(1, HIST):
                    acc = acc + bufs[k, t * HPAD + h, :]
                out_v[2 * q + t, :] = acc + bias
            nq = q + NBUF

            @pl.when(nq < qpw)
            def _():
                _gather(nq, k)
        return carry

    lax.fori_loop(0, qpw // NBUF, g_body, 0)

    pltpu.sync_copy(out_v, out_hbm.at[pl.ds(base, bpw)])


def _pool(p2, batch2, bias16):
    batch_size = batch2.shape[0] * 2
    bpw = batch_size // NW
    mesh = plsc.VectorSubcoreMesh(core_axis_name="c", subcore_axis_name="s")
    k = pl.kernel(
        functools.partial(_pool_body, bpw=bpw),
        out_type=jax.ShapeDtypeStruct((batch_size, LPAD), jnp.float32),
        mesh=mesh,
        scratch_types=[
            pltpu.VMEM((bpw // 2, 2 * HPAD), jnp.int32),
            pltpu.VMEM((NBUF, 2 * HPAD, LPAD), jnp.float32),
            pltpu.VMEM((bpw, LPAD), jnp.float32),
            pltpu.VMEM((LPAD,), jnp.float32),
            pltpu.SemaphoreType.DMA((NBUF,)),
        ],
        compiler_params=pltpu.CompilerParams(use_tc_tiling_on_sc=False),
    )
    return k(p2, batch2, bias16)


def kernel(batch, embed_weight, fc1_w, fc1_b):
    batch = batch.astype(jnp.int32)
    batch = jnp.pad(batch, ((0, 0), (0, HPAD - HIST)))
    labels = fc1_w.shape[1]
    w16 = jnp.pad(fc1_w, ((0, 0), (0, LPAD - labels)))
    b16 = jnp.pad(fc1_b, (0, LPAD - labels))
    w_big = (jnp.eye(NCOLB, dtype=jnp.float32)[:, None, :, None]
             * w16[None, :, None, :]).reshape(NCOLB * EMBED, NCOLB * LPAD)
    p = _pmat(embed_weight.T, w_big)
    p2 = p.reshape(VCHUNK * NCOLB, LPAD)
    batch2 = batch.reshape(batch.shape[0] // 2, 2 * HPAD)
    out16 = _pool(p2, batch2, b16)
    return out16[:, :labels]


# speedup vs baseline: 1.0480x; 1.0480x over previous
import functools

import jax
import jax.numpy as jnp
from jax import lax
from jax.experimental import pallas as pl
from jax.experimental.pallas import tpu as pltpu
from jax.experimental.pallas import tpu_sc as plsc

EMBED = 64
HIST = 50
HPAD = 64
HGAT = 56
NLANE = 16
LPAD = 16

NW = 32
NBUF = 8

NCOLB = 8
VCHUNK = 131072
MBLK = 8192


def _pmat_body(*refs):
    lhs = refs[:NCOLB]
    wb_ref, o_ref = refs[NCOLB], refs[NCOLB + 1]
    cat = jnp.concatenate([lhs[j][...] for j in range(NCOLB)], axis=0)
    o_ref[...] = lax.dot_general(
        cat, wb_ref[...], (((0,), (0,)), ((), ())),
        preferred_element_type=jnp.float32,
    )


def _pmat(table_t, w_big):
    grid = VCHUNK // MBLK
    nblk = table_t.shape[1] // MBLK
    in_specs = [
        pl.BlockSpec((EMBED, MBLK),
                     functools.partial(
                         lambda i, j: (0, jnp.minimum((VCHUNK // MBLK) * j + i,
                                                      nblk)),
                         j=j))
        for j in range(NCOLB)
    ]
    in_specs.append(
        pl.BlockSpec((NCOLB * EMBED, NCOLB * LPAD), lambda i: (0, 0)))
    return pl.pallas_call(
        _pmat_body,
        grid=(grid,),
        in_specs=in_specs,
        out_specs=pl.BlockSpec((MBLK, NCOLB * LPAD), lambda i: (i, 0)),
        out_shape=jax.ShapeDtypeStruct((VCHUNK, NCOLB * LPAD), jnp.float32),
    )(*([table_t] * NCOLB), w_big)


def _pool_body(p_hbm, batch_hbm, bias_hbm, out_hbm, idx_v, bufs, out_v,
               bias_v, sems, *, bpw):
    wid = lax.axis_index("s") * 2 + lax.axis_index("c")
    base = wid * bpw

    pltpu.sync_copy(batch_hbm.at[pl.ds(base, bpw)], idx_v)
    pltpu.sync_copy(bias_hbm, bias_v)

    lanes = lax.iota(jnp.int32, NLANE)

    def remap(b, carry):
        r0 = None
        for j in range(HPAD // NLANE):
            v = idx_v[b, pl.ds(j * NLANE, NLANE)]
            r = ((v & (VCHUNK - 1)) << 3) | (v >> 17)
            if j == 0:
                r0 = r
            if j == HPAD // NLANE - 1:
                r = jnp.where(lanes < (HIST - (HPAD - NLANE)), r, r0)
            idx_v[b, pl.ds(j * NLANE, NLANE)] = r
        return carry

    lax.fori_loop(0, bpw, remap, 0)

    def _gather(b, k):
        pltpu.make_async_copy(
            p_hbm.at[idx_v.at[b, pl.ds(0, HGAT)]], bufs.at[k],
            sems.at[k]).start()

    def _wait(b, k):
        pltpu.make_async_copy(
            p_hbm.at[idx_v.at[b, pl.ds(0, HGAT)]], bufs.at[k],
            sems.at[k]).wait()

    for k in range(NBUF):
        _gather(k, k)

    bias = bias_v[...]

    def g_body(g, carry):
        for k in range(NBUF):
            b = g * NBUF + k
            _wait(b, k)
            acc = bufs[k, 0, :]
            for h in range(1, HIST):
                acc = acc + bufs[k, h, :]
            out_v[b, :] = acc + bias
            nb = b + NBUF

            @pl.when(nb < bpw)
            def _():
                _gather(nb, k)
        return carry

    lax.fori_loop(0, bpw // NBUF, g_body, 0)

    pltpu.sync_copy(out_v, out_hbm.at[pl.ds(base, bpw)])


def _pool(p2, batch, bias16):
    batch_size = batch.shape[0]
    bpw = batch_size // NW
    mesh = plsc.VectorSubcoreMesh(core_axis_name="c", subcore_axis_name="s")
    k = pl.kernel(
        functools.partial(_pool_body, bpw=bpw),
        out_type=jax.ShapeDtypeStruct((batch_size, LPAD), jnp.float32),
        mesh=mesh,
        scratch_types=[
            pltpu.VMEM((bpw, HPAD), jnp.int32),
            pltpu.VMEM((NBUF, HGAT, LPAD), jnp.float32),
            pltpu.VMEM((bpw, LPAD), jnp.float32),
            pltpu.VMEM((LPAD,), jnp.float32),
            pltpu.SemaphoreType.DMA((NBUF,)),
        ],
        compiler_params=pltpu.CompilerParams(use_tc_tiling_on_sc=False),
    )
    return k(p2, batch, bias16)


def kernel(batch, embed_weight, fc1_w, fc1_b):
    batch = batch.astype(jnp.int32)
    batch = jnp.pad(batch, ((0, 0), (0, HPAD - HIST)))
    labels = fc1_w.shape[1]
    w16 = jnp.pad(fc1_w, ((0, 0), (0, LPAD - labels)))
    b16 = jnp.pad(fc1_b, (0, LPAD - labels))
    w_big = (jnp.eye(NCOLB, dtype=jnp.float32)[:, None, :, None]
             * w16[None, :, None, :]).reshape(NCOLB * EMBED, NCOLB * LPAD)
    p = _pmat(embed_weight.T, w_big)
    p2 = p.reshape(VCHUNK * NCOLB, LPAD)
    out16 = _pool(p2, batch, b16)
    return out16[:, :labels]


# speedup vs baseline: 1.0485x; 1.0005x over previous
import functools

import jax
import jax.numpy as jnp
from jax import lax
from jax.experimental import pallas as pl
from jax.experimental.pallas import tpu as pltpu
from jax.experimental.pallas import tpu_sc as plsc

EMBED = 64
HIST = 50
HPAD = 64
HGAT = 56
NLANE = 16
LPAD = 16

NW = 32
NBUF = 8

NCOLB = 8
VCHUNK = 131072
MBLK = 8192


def _pmat_body(*refs):
    lhs = refs[:NCOLB]
    wb_ref, o_ref = refs[NCOLB], refs[NCOLB + 1]
    cat = jnp.concatenate(
        [lhs[j][...].astype(jnp.bfloat16) for j in range(NCOLB)], axis=0)
    o_ref[...] = lax.dot_general(
        cat, wb_ref[...], (((0,), (0,)), ((), ())),
        preferred_element_type=jnp.float32,
    )


def _pmat(table_t, w_big):
    grid = VCHUNK // MBLK
    nblk = table_t.shape[1] // MBLK
    in_specs = [
        pl.BlockSpec((EMBED, MBLK),
                     functools.partial(
                         lambda i, j: (0, jnp.minimum((VCHUNK // MBLK) * j + i,
                                                      nblk)),
                         j=j))
        for j in range(NCOLB)
    ]
    in_specs.append(
        pl.BlockSpec((NCOLB * EMBED, NCOLB * LPAD), lambda i: (0, 0)))
    return pl.pallas_call(
        _pmat_body,
        grid=(grid,),
        in_specs=in_specs,
        out_specs=pl.BlockSpec((MBLK, NCOLB * LPAD), lambda i: (i, 0)),
        out_shape=jax.ShapeDtypeStruct((VCHUNK, NCOLB * LPAD), jnp.float32),
    )(*([table_t] * NCOLB), w_big)


def _pool_body(p_hbm, batch_hbm, bias_hbm, out_hbm, idx_v, bufs, out_v,
               bias_v, sems, *, bpw):
    wid = lax.axis_index("s") * 2 + lax.axis_index("c")
    base = wid * bpw

    pltpu.sync_copy(batch_hbm.at[pl.ds(base, bpw)], idx_v)
    pltpu.sync_copy(bias_hbm, bias_v)

    lanes = lax.iota(jnp.int32, NLANE)

    def remap(b, carry):
        r0 = None
        for j in range(HPAD // NLANE):
            v = idx_v[b, pl.ds(j * NLANE, NLANE)]
            r = ((v & (VCHUNK - 1)) << 3) | (v >> 17)
            if j == 0:
                r0 = r
            if j == HPAD // NLANE - 1:
                r = jnp.where(lanes < (HIST - (HPAD - NLANE)), r, r0)
            idx_v[b, pl.ds(j * NLANE, NLANE)] = r
        return carry

    lax.fori_loop(0, bpw, remap, 0)

    def _gather(b, k):
        pltpu.make_async_copy(
            p_hbm.at[idx_v.at[b, pl.ds(0, HGAT)]], bufs.at[k],
            sems.at[k]).start()

    def _wait(b, k):
        pltpu.make_async_copy(
            p_hbm.at[idx_v.at[b, pl.ds(0, HGAT)]], bufs.at[k],
            sems.at[k]).wait()

    for k in range(NBUF):
        _gather(k, k)

    bias = bias_v[...]

    def g_body(g, carry):
        for k in range(NBUF):
            b = g * NBUF + k
            _wait(b, k)
            acc = bufs[k, 0, :]
            for h in range(1, HIST):
                acc = acc + bufs[k, h, :]
            out_v[b, :] = acc + bias
            nb = b + NBUF

            @pl.when(nb < bpw)
            def _():
                _gather(nb, k)
        return carry

    lax.fori_loop(0, bpw // NBUF, g_body, 0)

    pltpu.sync_copy(out_v, out_hbm.at[pl.ds(base, bpw)])


def _pool(p2, batch, bias16):
    batch_size = batch.shape[0]
    bpw = batch_size // NW
    mesh = plsc.VectorSubcoreMesh(core_axis_name="c", subcore_axis_name="s")
    k = pl.kernel(
        functools.partial(_pool_body, bpw=bpw),
        out_type=jax.ShapeDtypeStruct((batch_size, LPAD), jnp.float32),
        mesh=mesh,
        scratch_types=[
            pltpu.VMEM((bpw, HPAD), jnp.int32),
            pltpu.VMEM((NBUF, HGAT, LPAD), jnp.float32),
            pltpu.VMEM((bpw, LPAD), jnp.float32),
            pltpu.VMEM((LPAD,), jnp.float32),
            pltpu.SemaphoreType.DMA((NBUF,)),
        ],
        compiler_params=pltpu.CompilerParams(use_tc_tiling_on_sc=False),
    )
    return k(p2, batch, bias16)


def kernel(batch, embed_weight, fc1_w, fc1_b):
    batch = batch.astype(jnp.int32)
    batch = jnp.pad(batch, ((0, 0), (0, HPAD - HIST)))
    labels = fc1_w.shape[1]
    w16 = jnp.pad(fc1_w, ((0, 0), (0, LPAD - labels)))
    b16 = jnp.pad(fc1_b, (0, LPAD - labels))
    w_big = (jnp.eye(NCOLB, dtype=jnp.float32)[:, None, :, None]
             * w16[None, :, None, :]).reshape(
                 NCOLB * EMBED, NCOLB * LPAD).astype(jnp.bfloat16)
    p = _pmat(embed_weight.T, w_big)
    p2 = p.reshape(VCHUNK * NCOLB, LPAD)
    out16 = _pool(p2, batch, b16)
    return out16[:, :labels]
